# Initial kernel scaffold; baseline (speedup 1.0000x reference)
import jax
import jax.numpy as jnp
from jax.experimental import pallas as pl


def kernel(x, edge_index, batch, d, d_index, W1, b1, W2, b2, W3, b3, Wf1, bf1, Wf2, bf2, Wf3, bf3):
    raise NotImplementedError("write your pallas kernel here")


# speedup vs baseline: 23.1930x; 23.1930x over previous
import functools

import jax
import jax.numpy as jnp
from jax.experimental import pallas as pl
from jax.experimental.pallas import tpu as pltpu

N = 10000
E = 160000
F_IN = 256
H = 256
LEV = 2
R_ = 3
NS = (R_ - 1) * LEV + 1
B = 32
NNZ = 800000
C = 10
NPAD = 10240


def _i0():
    return jnp.asarray(0, jnp.int32)


def _mm_body(x_ref, w_ref, o_ref):
    o_ref[...] = jnp.dot(x_ref[...], w_ref[...],
                         preferred_element_type=jnp.float32)


def _matmul(x, w, bm=2048):
    M, K = x.shape
    F = w.shape[1]
    assert M % bm == 0
    return pl.pallas_call(
        _mm_body,
        grid=(M // bm,),
        in_specs=[pl.BlockSpec((bm, K), lambda i: (i, _i0())),
                  pl.BlockSpec((K, F), lambda i: (_i0(), _i0())),
        ],
        out_specs=pl.BlockSpec((bm, F), lambda i: (i, _i0())),
        out_shape=jax.ShapeDtypeStruct((M, F), jnp.float32),
    )(x, w)


def _head_body(pm_ref, xc_ref, wf1_ref, bf1_ref, wf2_ref, bf2_ref,
               wf3_ref, bf3_ref, o_ref):
    pooled = jnp.dot(pm_ref[...], xc_ref[...],
                     preferred_element_type=jnp.float32)
    g = pooled.reshape(B, NS * 3 * H)
    h = jax.nn.relu(jnp.dot(g, wf1_ref[...],
                            preferred_element_type=jnp.float32) + bf1_ref[...])
    h = jax.nn.relu(jnp.dot(h, wf2_ref[...],
                            preferred_element_type=jnp.float32) + bf2_ref[...])
    logits = jnp.dot(h, wf3_ref[...],
                     preferred_element_type=jnp.float32) + bf3_ref[...]
    colid = jax.lax.broadcasted_iota(jnp.int32, logits.shape, 1)
    masked = jnp.where(colid < C, logits, -jnp.inf)
    mx = jnp.max(masked, axis=-1, keepdims=True)
    lse = jnp.log(jnp.sum(jnp.where(colid < C, jnp.exp(masked - mx), 0.0),
                          axis=-1, keepdims=True)) + mx
    o_ref[...] = jnp.where(colid < C, masked - lse, 0.0)


def _head(pmat, xc, wf1, bf1, wf2, bf2, wf3p, bf3p):
    full = lambda shape: pl.BlockSpec(shape, lambda: tuple(_i0() for _ in shape))
    return pl.pallas_call(
        _head_body,
        in_specs=[full((B * NS, NPAD)), full((NPAD, 3 * H)),
                  full((NS * 3 * H, 3 * H)), full((3 * H,)),
                  full((3 * H, H)), full((H,)),
                  full((H, 128)), full((128,))],
        out_specs=full((B, 128)),
        out_shape=jax.ShapeDtypeStruct((B, 128), jnp.float32),
    )(pmat, xc, wf1, bf1, wf2, bf2, wf3p, bf3p)


def kernel(x, edge_index, batch, d, d_index, W1, b1, W2, b2, W3, b3,
           Wf1, bf1, Wf2, bf2, Wf3, bf3):
    out_dtype = jnp.result_type(x.dtype, W1.dtype)
    x = x.astype(jnp.float32)
    d = d.astype(jnp.float32)
    W1, b1, W2, b2, W3, b3 = (a.astype(jnp.float32) for a in (W1, b1, W2, b2, W3, b3))
    Wf1, bf1, Wf2, bf2, Wf3, bf3 = (a.astype(jnp.float32) for a in (Wf1, bf1, Wf2, bf2, Wf3, bf3))
    src = edge_index[0].astype(jnp.int32)
    dst = edge_index[1].astype(jnp.int32)
    batch32 = batch.astype(jnp.int32)
    row = d_index[0].astype(jnp.int32)
    col0 = d_index[1].astype(jnp.int32)

    deg = jnp.zeros((N,), jnp.float32).at[dst].add(1.0) + 1.0
    dinv = jax.lax.rsqrt(jnp.clip(deg, 1.0, None))
    dinv2 = dinv * dinv
    norm = dinv[src] * dinv[dst]

    xpad = jnp.pad(x, ((0, NPAD - N), (0, 0)))

    def gcn(h, W, b):
        hw = _matmul(h, W)
        agg = jnp.zeros((N, H), jnp.float32).at[dst].add(
            norm[:, None] * hw[src])
        out = agg + dinv2[:, None] * hw[:N] + b
        out = jax.nn.relu(out)
        return jnp.pad(out, ((0, NPAD - N), (0, 0)))

    x1 = gcn(xpad, W1, b1)
    x2 = gcn(x1, W2, b2)
    x3 = gcn(x2, W3, b3)
    xc = jnp.concatenate([x1, x2, x3], axis=-1)

    s = row // N
    n_ = row - s * N
    col = col0 % N
    seg = batch32[n_] * NS + s
    pmat = jnp.zeros((B * NS, NPAD), jnp.float32).at[seg, col].add(d)

    wf3p = jnp.pad(Wf3, ((0, 0), (0, 128 - C)))
    bf3p = jnp.pad(bf3, ((0, 128 - C),))
    out = _head(pmat, xc, Wf1, bf1, Wf2, bf2, wf3p, bf3p)
    return out[:, :C].astype(out_dtype)


# speedup vs baseline: 39.6320x; 1.7088x over previous
import functools

import jax
import jax.numpy as jnp
from jax import lax
from jax.experimental import pallas as pl
from jax.experimental.pallas import tpu as pltpu
from jax.experimental.pallas import tpu_sc as plsc

N = 10000
E = 160000
F_IN = 256
H = 256
LEV = 2
R_ = 3
NS = (R_ - 1) * LEV + 1
B = 32
NNZ = 800000
C = 10
NPAD = 10240


def _i0():
    return jnp.asarray(0, jnp.int32)


def _mm_body(x_ref, w_ref, o_ref):
    o_ref[...] = jnp.dot(x_ref[...], w_ref[...],
                         preferred_element_type=jnp.float32)


def _matmul(x, w, bm=2048):
    M, K = x.shape
    F = w.shape[1]
    assert M % bm == 0
    return pl.pallas_call(
        _mm_body,
        grid=(M // bm,),
        in_specs=[pl.BlockSpec((bm, K), lambda i: (i, _i0())),
                  pl.BlockSpec((K, F), lambda i: (_i0(), _i0())),
        ],
        out_specs=pl.BlockSpec((bm, F), lambda i: (i, _i0())),
        out_shape=jax.ShapeDtypeStruct((M, F), jnp.float32),
    )(x, w)


def _head_body(pm_ref, xc_ref, wf1_ref, bf1_ref, wf2_ref, bf2_ref,
               wf3_ref, bf3_ref, o_ref):
    pooled = jnp.dot(pm_ref[...], xc_ref[...],
                     preferred_element_type=jnp.float32)
    g = pooled.reshape(B, NS * 3 * H)
    h = jax.nn.relu(jnp.dot(g, wf1_ref[...],
                            preferred_element_type=jnp.float32) + bf1_ref[...])
    h = jax.nn.relu(jnp.dot(h, wf2_ref[...],
                            preferred_element_type=jnp.float32) + bf2_ref[...])
    logits = jnp.dot(h, wf3_ref[...],
                     preferred_element_type=jnp.float32) + bf3_ref[...]
    colid = jax.lax.broadcasted_iota(jnp.int32, logits.shape, 1)
    masked = jnp.where(colid < C, logits, -jnp.inf)
    mx = jnp.max(masked, axis=-1, keepdims=True)
    lse = jnp.log(jnp.sum(jnp.where(colid < C, jnp.exp(masked - mx), 0.0),
                          axis=-1, keepdims=True)) + mx
    o_ref[...] = jnp.where(colid < C, masked - lse, 0.0)


def _head(pmat, xc, wf1, bf1, wf2, bf2, wf3p, bf3p):
    full = lambda shape: pl.BlockSpec(shape, lambda: tuple(_i0() for _ in shape))
    return pl.pallas_call(
        _head_body,
        in_specs=[full((B * NS, NPAD)), full((NPAD, 3 * H)),
                  full((NS * 3 * H, 3 * H)), full((3 * H,)),
                  full((3 * H, H)), full((H,)),
                  full((H, 128)), full((128,))],
        out_specs=full((B, 128)),
        out_shape=jax.ShapeDtypeStruct((B, 128), jnp.float32),
    )(pmat, xc, wf1, bf1, wf2, bf2, wf3p, bf3p)


_SC_MESH = plsc.VectorSubcoreMesh(core_axis_name="c", subcore_axis_name="s")
PB_CH = 2048
PROWS = 5


def _c(v):
    return jnp.asarray(v, jnp.int32)


def _fori(n, body, init=0):
    return lax.fori_loop(_c(0), _c(n), body, init)


def _cv(v):
    return jnp.full((16,), v, jnp.int32)


def _divmod_n(v):
    q = (jnp.where(v >= _cv(N), _cv(1), _cv(0))
         + jnp.where(v >= _cv(2 * N), _cv(1), _cv(0))
         + jnp.where(v >= _cv(3 * N), _cv(1), _cv(0))
         + jnp.where(v >= _cv(4 * N), _cv(1), _cv(0)))
    return q, v - q * _cv(N)


def _pbuild_body(rows_hbm, cols_hbm, d_hbm, batch_hbm, p_hbm,
                 rows_v, cols_v, d_v, batch_v, acc_v):
    wid = lax.axis_index("s") * _c(2) + lax.axis_index("c")
    lo = wid * _c(PROWS)

    def zrow(r, carry):
        def zcol(j, c2):
            acc_v[r, pl.ds(j * _c(16), 16)] = jnp.zeros((16,), jnp.float32)
            return c2
        return _fori(NPAD // 16, zcol, carry)
    _fori(8, zrow, 0)

    pltpu.sync_copy(batch_hbm, batch_v)

    def chunk(ci, carry):
        base = ci * _c(PB_CH)
        pltpu.sync_copy(rows_hbm.at[pl.ds(base, PB_CH)], rows_v)
        pltpu.sync_copy(cols_hbm.at[pl.ds(base, PB_CH)], cols_v)
        pltpu.sync_copy(d_hbm.at[pl.ds(base, PB_CH)], d_v)

        def inner(k, c2):
            off = k * _c(16)
            rv = rows_v[pl.ds(off, 16)]
            cv = cols_v[pl.ds(off, 16)]
            dv = d_v[pl.ds(off, 16)]
            s, n_ = _divmod_n(rv)
            _, c = _divmod_n(cv)
            b = plsc.load_gather(batch_v, [n_])
            local = b * _cv(NS) + s - jnp.broadcast_to(lo, (16,))
            msk = (local >= _cv(0)) & (local < _cv(PROWS))
            local = jnp.where(msk, local, _cv(0))
            plsc.addupdate_scatter(acc_v, [local, c], dv, mask=msk)
            return c2
        return _fori(PB_CH // 16, inner, carry)
    _fori(NNZ // PB_CH, chunk, 0)

    pltpu.sync_copy(acc_v, p_hbm.at[wid])


def _pbuild(rows, cols, dvals, batchp):
    f = pl.kernel(
        _pbuild_body,
        out_type=jax.ShapeDtypeStruct((32, 8, NPAD), jnp.float32),
        mesh=_SC_MESH,
        compiler_params=pltpu.CompilerParams(needs_layout_passes=False),
        scratch_types=[
            pltpu.VMEM((PB_CH,), jnp.int32),
            pltpu.VMEM((PB_CH,), jnp.int32),
            pltpu.VMEM((PB_CH,), jnp.float32),
            pltpu.VMEM((NPAD,), jnp.int32),
            pltpu.VMEM((8, NPAD), jnp.float32),
        ],
    )
    out3 = f(rows, cols, dvals, batchp)
    return out3[:, :PROWS, :].reshape(B * NS, NPAD)


def kernel(x, edge_index, batch, d, d_index, W1, b1, W2, b2, W3, b3,
           Wf1, bf1, Wf2, bf2, Wf3, bf3):
    out_dtype = jnp.result_type(x.dtype, W1.dtype)
    x = x.astype(jnp.float32)
    d = d.astype(jnp.float32)
    W1, b1, W2, b2, W3, b3 = (a.astype(jnp.float32) for a in (W1, b1, W2, b2, W3, b3))
    Wf1, bf1, Wf2, bf2, Wf3, bf3 = (a.astype(jnp.float32) for a in (Wf1, bf1, Wf2, bf2, Wf3, bf3))
    src = edge_index[0].astype(jnp.int32)
    dst = edge_index[1].astype(jnp.int32)
    batch32 = batch.astype(jnp.int32)
    row = d_index[0].astype(jnp.int32)
    col0 = d_index[1].astype(jnp.int32)

    deg = jnp.zeros((N,), jnp.float32).at[dst].add(1.0) + 1.0
    dinv = jax.lax.rsqrt(jnp.clip(deg, 1.0, None))
    dinv2 = dinv * dinv
    norm = dinv[src] * dinv[dst]

    xpad = jnp.pad(x, ((0, NPAD - N), (0, 0)))

    def gcn(h, W, b):
        hw = _matmul(h, W)
        agg = jnp.zeros((N, H), jnp.float32).at[dst].add(
            norm[:, None] * hw[src])
        out = agg + dinv2[:, None] * hw[:N] + b
        out = jax.nn.relu(out)
        return jnp.pad(out, ((0, NPAD - N), (0, 0)))

    x1 = gcn(xpad, W1, b1)
    x2 = gcn(x1, W2, b2)
    x3 = gcn(x2, W3, b3)
    xc = jnp.concatenate([x1, x2, x3], axis=-1)

    batchp = jnp.pad(batch32, ((0, NPAD - N),))
    pmat = _pbuild(row, col0, d, batchp)

    wf3p = jnp.pad(Wf3, ((0, 0), (0, 128 - C)))
    bf3p = jnp.pad(bf3, ((0, 128 - C),))
    out = _head(pmat, xc, Wf1, bf1, Wf2, bf2, wf3p, bf3p)
    return out[:, :C].astype(out_dtype)
